# Initial kernel scaffold; baseline (speedup 1.0000x reference)
import jax
import jax.numpy as jnp
from jax.experimental import pallas as pl


def kernel(features, edge_index, W1, b1, W2, b2, W3, b3, Wfc, bfc):
    raise NotImplementedError("write your pallas kernel here")


# speedup vs baseline: 1.7827x; 1.7827x over previous
import functools

import jax
import jax.numpy as jnp
from jax import lax
from jax.experimental import pallas as pl
from jax.experimental.pallas import tpu as pltpu
from jax.experimental.pallas import tpu_sc as plsc

NODES = 10000
M_PAD = 10240
ZROWS = M_PAD - NODES
EDGES = 160000
E_PAD = 163840
NC, NS = 2, 16
NW = NC * NS
TPB = M_PAD // NW
CAP = 16384
SCN = 512
CHUNK = 32
DD = 256
BM = 1024

_SC_PARAMS = pltpu.CompilerParams(needs_layout_passes=False)


def _mesh():
    return plsc.VectorSubcoreMesh(core_axis_name="c", subcore_axis_name="s")


@functools.lru_cache(maxsize=None)
def _make_part():
    @functools.partial(
        pl.kernel,
        mesh=_mesh(),
        out_type=(jax.ShapeDtypeStruct((NW * CAP,), jnp.int32),
                  jax.ShapeDtypeStruct((NW * 16,), jnp.int32),
                  jax.ShapeDtypeStruct((M_PAD,), jnp.float32)),
        compiler_params=_SC_PARAMS,
        scratch_types=[
            pltpu.VMEM((SCN,), jnp.int32),
            pltpu.VMEM((SCN,), jnp.int32),
            pltpu.VMEM((CAP + 16,), jnp.int32),
            pltpu.VMEM((TPB + 16,), jnp.float32),
            pltpu.VMEM((16,), jnp.int32),
        ],
    )
    def part(gidx, sidx, packed, cnts, deg, gst, sst, pbuf, dacc, cbuf):
        cid = lax.axis_index("c")
        sid = lax.axis_index("s")
        w = cid * NS + sid
        lo = w * TPB
        io = lax.iota(jnp.int32, 16)

        def chunk_body(c, pos):
            pltpu.sync_copy(gidx.at[pl.ds(c * SCN, SCN)], gst)
            pltpu.sync_copy(sidx.at[pl.ds(c * SCN, SCN)], sst)

            def grp(t, pos):
                sl = pl.ds(t * 16, 16)
                g = gst[sl]
                s = sst[sl]
                m = (s >= lo) & (s < lo + TPB)
                pk = lax.shift_left(s - lo, 18) | g
                plsc.store_compressed(pbuf.at[pl.ds(pos, 16)], pk, mask=m)
                pos = pos + plsc.all_reduce_population_count(m)[0]
                return jnp.minimum(pos, CAP - 48)

            return lax.fori_loop(0, SCN // 16, grp, pos)

        pos = lax.fori_loop(0, E_PAD // SCN, chunk_body, 0)

        def safe(h):
            return lax.shift_left(lax.rem(h, TPB), 18) | (NODES + lax.rem(h, ZROWS))

        h0 = pos + io * 13 + w * 37
        pbuf[pl.ds(pos, 16)] = safe(h0)
        pbuf[pl.ds(pos + 16, 16)] = safe(h0 + 7)
        cpad = ((pos + 31) // 32) * 32

        def zdeg(r, _):
            dacc[pl.ds(r * 16, 16)] = jnp.zeros((16,), jnp.float32)
            return 0

        lax.fori_loop(0, (TPB + 16) // 16, zdeg, 0)
        e0 = jnp.where(io == 0, 1.0, 0.0)

        def hist(e, _):
            r = lax.shift_right_logical(pbuf[pl.ds(e, 16)][0], 18)
            dacc[pl.ds(r, 16)] = dacc[pl.ds(r, 16)] + e0
            return 0

        lax.fori_loop(0, pos, hist, 0)

        pltpu.sync_copy(pbuf.at[pl.ds(0, CAP)], packed.at[pl.ds(w * CAP, CAP)])
        cbuf[pl.ds(0, 16)] = jnp.zeros((16,), jnp.int32) + cpad
        pltpu.sync_copy(cbuf, cnts.at[pl.ds(w * 16, 16)])
        pltpu.sync_copy(dacc.at[pl.ds(0, TPB)], deg.at[pl.ds(w * TPB, TPB)])

    return part


@functools.lru_cache(maxsize=None)
def _make_agg():
    @functools.partial(
        pl.kernel,
        mesh=_mesh(),
        out_type=jax.ShapeDtypeStruct((M_PAD, DD), jnp.float32),
        compiler_params=_SC_PARAMS,
        scratch_types=[
            pltpu.VMEM((2, CHUNK), jnp.int32),
            pltpu.VMEM((2, CHUNK), jnp.int32),
            pltpu.VMEM((2, CHUNK + 16), jnp.int32),
            pltpu.VMEM((2, CHUNK, DD), jnp.float32),
            pltpu.VMEM((TPB, DD), jnp.float32),
            pltpu.VMEM((16,), jnp.int32),
            pltpu.SemaphoreType.DMA,
            pltpu.SemaphoreType.DMA,
        ],
    )
    def agg(tbl, packed, cnts, out, pkb, gib, dib, rows, acc, cb, sem0, sem1):
        cid = lax.axis_index("c")
        sid = lax.axis_index("s")
        w = cid * NS + sid
        sems = (sem0, sem1)

        pltpu.sync_copy(cnts.at[pl.ds(w * 16, 16)], cb)
        nch = cb[pl.ds(0, 16)][0] // CHUNK

        def zr(r, _):
            for j in range(DD // 16):
                acc[r, pl.ds(j * 16, 16)] = jnp.zeros((16,), jnp.float32)
            return 0

        lax.fori_loop(0, TPB, zr, 0)

        def stage(c, b):
            pltpu.sync_copy(packed.at[pl.ds(w * CAP + c * CHUNK, CHUNK)],
                            pkb.at[b])
            for t in range(CHUNK // 16):
                sl = pl.ds(t * 16, 16)
                v = pkb[b, sl]
                gib[b, sl] = v & 0x3FFFF
                dib[b, sl] = lax.shift_right_logical(v, 18)
            pltpu.async_copy(tbl.at[gib.at[b]], rows.at[b], sems[b])

        def accum(b):
            pltpu.make_async_copy(tbl.at[gib.at[b]], rows.at[b], sems[b]).wait()

            def per_edge(e, _):
                r = dib[b, pl.ds(e, 16)][0]
                for j in range(DD // 16):
                    sl = pl.ds(j * 16, 16)
                    acc[r, sl] = acc[r, sl] + rows[b, e, sl]
                return 0

            lax.fori_loop(0, CHUNK, per_edge, 0)

        @pl.when(nch > 0)
        def _():
            stage(0, 0)

        def body(g, _):
            c = g * 2

            @pl.when(c + 1 < nch)
            def _():
                stage(c + 1, 1)

            accum(0)

            @pl.when(c + 2 < nch)
            def _():
                stage(c + 2, 0)

            @pl.when(c + 1 < nch)
            def _():
                accum(1)

            return 0

        lax.fori_loop(0, (nch + 1) // 2, body, 0)
        pltpu.sync_copy(acc, out.at[pl.ds(w * TPB, TPB)])

    return agg


def _norm(deg_col):
    return jnp.where(deg_col >= 1.0, lax.rsqrt(jnp.maximum(deg_col, 1.0)), 1.0)


def _mm_first_body(a_ref, w_ref, dgo_ref, o_ref):
    h = jnp.dot(a_ref[...], w_ref[...], preferred_element_type=jnp.float32)
    o_ref[...] = h * _norm(dgo_ref[...])


def _mm_mid_body(a_ref, dgi_ref, b_ref, w_ref, dgo_ref, o_ref):
    i = pl.program_id(0)
    x = jnp.maximum(a_ref[...] * _norm(dgi_ref[...]) + b_ref[...], 0.0)
    rows = i * BM + lax.broadcasted_iota(jnp.int32, (BM, 1), 0)
    x = jnp.where(rows < NODES, x, 0.0)
    h = jnp.dot(x, w_ref[...], preferred_element_type=jnp.float32)
    o_ref[...] = h * _norm(dgo_ref[...])


def _mm_fin_body(a_ref, dgi_ref, b_ref, w_ref, bf_ref, o_ref):
    x = jnp.maximum(a_ref[...] * _norm(dgi_ref[...]) + b_ref[...], 0.0)
    o_ref[...] = jnp.dot(x, w_ref[...], preferred_element_type=jnp.float32) + bf_ref[...]


def _row_spec(d):
    return pl.BlockSpec((BM, d), lambda i: (i, 0))


def _fix_spec(shape):
    return pl.BlockSpec(shape, lambda i: (0, 0))


def _mm_first(a, w, dgo):
    k, n = w.shape
    return pl.pallas_call(
        _mm_first_body,
        grid=(M_PAD // BM,),
        in_specs=[_row_spec(k), _fix_spec((k, n)), _row_spec(1)],
        out_specs=_row_spec(n),
        out_shape=jax.ShapeDtypeStruct((M_PAD, n), jnp.float32),
    )(a, w, dgo)


def _mm_mid(a, dgi, b, w, dgo):
    k, n = w.shape
    return pl.pallas_call(
        _mm_mid_body,
        grid=(M_PAD // BM,),
        in_specs=[_row_spec(k), _row_spec(1), _fix_spec((1, k)),
                  _fix_spec((k, n)), _row_spec(1)],
        out_specs=_row_spec(n),
        out_shape=jax.ShapeDtypeStruct((M_PAD, n), jnp.float32),
    )(a, dgi, b, w, dgo)


def _mm_fin(a, dgi, b, w, bf):
    k, n = w.shape
    return pl.pallas_call(
        _mm_fin_body,
        grid=(M_PAD // BM,),
        in_specs=[_row_spec(k), _row_spec(1), _fix_spec((1, k)),
                  _fix_spec((k, n)), _fix_spec((1, n))],
        out_specs=_row_spec(n),
        out_shape=jax.ShapeDtypeStruct((M_PAD, n), jnp.float32),
    )(a, dgi, b, w, bf)


def kernel(features, edge_index, W1, b1, W2, b2, W3, b3, Wfc, bfc):
    ei = edge_index.astype(jnp.int32)
    pad = jnp.full((E_PAD - EDGES,), NODES, dtype=jnp.int32)
    src = jnp.concatenate([ei[0], pad])
    dst = jnp.concatenate([ei[1], pad])

    feats = jnp.pad(features, ((0, M_PAD - NODES), (0, 1)))
    w1 = jnp.pad(W1, ((0, 1), (0, 0)))
    wfc = jnp.pad(Wfc, ((0, 0), (0, 118)))
    bfc_p = jnp.pad(bfc, (0, 118)).reshape(1, -1)

    part = _make_part()
    pk, cnt, deg_in = part(src, dst)
    _pk2, _cnt2, deg_out = part(dst, src)
    dgi = deg_in.reshape(M_PAD, 1)
    dgo = deg_out.reshape(M_PAD, 1)

    agg = _make_agg()
    hs1 = _mm_first(feats, w1, dgo)
    agg1 = agg(hs1, pk, cnt)
    hs2 = _mm_mid(agg1, dgi, b1.reshape(1, -1), W2, dgo)
    agg2 = agg(hs2, pk, cnt)
    hs3 = _mm_mid(agg2, dgi, b2.reshape(1, -1), W3, dgo)
    agg3 = agg(hs3, pk, cnt)
    out = _mm_fin(agg3, dgi, b3.reshape(1, -1), wfc, bfc_p)
    return out[:NODES, :10]
